# Initial kernel scaffold; baseline (speedup 1.0000x reference)
import jax
import jax.numpy as jnp
from jax.experimental import pallas as pl


def kernel(x_embed, s_prompt_key, t_prompt_key, m_prompt_key):
    raise NotImplementedError("write your pallas kernel here")


# speedup vs baseline: 5.9929x; 5.9929x over previous
import jax
import jax.numpy as jnp
from jax.experimental import pallas as pl
from jax.experimental.pallas import tpu as pltpu

_B, _S, _D = 1024, 128, 768
_POOL = 4096
_TOP_K = 16

_EB = 32
_BB = 128


def _embed_kernel(x_ref, o_ref):
    x = x_ref[...]
    m = jnp.sum(x, axis=1) * (1.0 / _S)
    ss = jnp.sum(m * m, axis=-1, keepdims=True)
    o_ref[...] = m * jax.lax.rsqrt(jnp.maximum(ss, 1e-12))


def _sim_topk_kernel(xn_ref, key_ref, sim_ref, topv_ref, topi_ref, kn_ref):
    @pl.when(pl.program_id(0) == 0)
    def _normalize_keys():
        k = key_ref[...]
        ss = jnp.sum(k * k, axis=-1, keepdims=True)
        kn_ref[...] = k * jax.lax.rsqrt(jnp.maximum(ss, 1e-12))

    sim = jax.lax.dot_general(
        xn_ref[...], kn_ref[...],
        dimension_numbers=(((1,), (1,)), ((), ())),
        preferred_element_type=jnp.float32)
    sim_ref[...] = sim

    w = sim
    iota = jax.lax.broadcasted_iota(jnp.int32, sim.shape, 1)
    vals, idxs = [], []
    for _ in range(_TOP_K):
        m = jnp.max(w, axis=1)
        idx = jnp.min(jnp.where(w == m[:, None], iota, _POOL), axis=1)
        vals.append(m)
        idxs.append(idx)
        w = jnp.where(iota == idx[:, None], -jnp.inf, w)
    topv_ref[...] = jnp.stack(vals, axis=1)
    topi_ref[...] = jnp.stack(idxs, axis=1)


def _pool_sim_topk(xn, key):
    return pl.pallas_call(
        _sim_topk_kernel,
        grid=(_B // _BB,),
        in_specs=[
            pl.BlockSpec((_BB, _D), lambda i: (i, 0)),
            pl.BlockSpec((_POOL, _D), lambda i: (0, 0)),
        ],
        out_specs=[
            pl.BlockSpec((_BB, _POOL), lambda i: (i, 0)),
            pl.BlockSpec((_BB, _TOP_K), lambda i: (i, 0)),
            pl.BlockSpec((_BB, _TOP_K), lambda i: (i, 0)),
        ],
        out_shape=[
            jax.ShapeDtypeStruct((_B, _POOL), jnp.float32),
            jax.ShapeDtypeStruct((_B, _TOP_K), jnp.float32),
            jax.ShapeDtypeStruct((_B, _TOP_K), jnp.int32),
        ],
        scratch_shapes=[pltpu.VMEM((_POOL, _D), jnp.float32)],
    )(xn, key)


def kernel(x_embed, s_prompt_key, t_prompt_key, m_prompt_key):
    xn = pl.pallas_call(
        _embed_kernel,
        grid=(_B // _EB,),
        in_specs=[pl.BlockSpec((_EB, _S, _D), lambda i: (i, 0, 0))],
        out_specs=pl.BlockSpec((_EB, _D), lambda i: (i, 0)),
        out_shape=jax.ShapeDtypeStruct((_B, _D), jnp.float32),
    )(x_embed)

    s_sim, s_v, s_i = _pool_sim_topk(xn, s_prompt_key)
    t_sim, t_v, t_i = _pool_sim_topk(xn, t_prompt_key)
    m_sim, m_v, m_i = _pool_sim_topk(xn, m_prompt_key)
    return (s_sim, s_v, s_i, t_sim, t_v, t_i, m_sim, m_v, m_i)
